# baseline (speedup 1.0000x reference)
import jax
import jax.numpy as jnp
from jax.experimental import pallas as pl
from jax.experimental.pallas import tpu as pltpu

H, W = 4096, 4096
N_BINS = 256
D = 255

LANES = 128
ROWS_FLAT = H * W // LANES
BLK_ROWS = 1024
HIST_STEPS = ROWS_FLAT // BLK_ROWS
PAIRS = BLK_ROWS // 16
NGRP = 32
SEG = 7

BIN_BLK_ROWS = 256
BIN_STEPS = H // BIN_BLK_ROWS


def _hist_kernel(x_ref, out_ref, byte_ref, wide_ref):
    j = pl.program_id(0)

    @pl.when(j == 0)
    def _():
        wide_ref[...] = jnp.zeros_like(wide_ref)

    byte_ref[...] = jnp.zeros_like(byte_ref)

    def pair_body(i, accs):
        x2 = x_ref[pl.ds(pl.multiple_of(i * 16, 8), 16), :]
        grp = x2 >> 3
        t = jnp.int32(1) << ((x2 & 7) << 2)
        out = []
        for g in range(NGRP):
            contrib = jnp.where(grp == g, t, 0)
            out.append(accs[g] + (contrib[:8, :] + contrib[8:, :]))
        return tuple(out)

    def flush_l1(accs):
        for g in range(NGRP):
            w = accs[g]
            byte_ref[2 * g] += w & 0x0F0F0F0F
            byte_ref[2 * g + 1] += (w >> 4) & 0x0F0F0F0F

    zeros = tuple(jnp.zeros((8, LANES), jnp.int32) for _ in range(NGRP))
    base = 0
    for _ in range(9):
        accs = jax.lax.fori_loop(base, base + SEG, pair_body, zeros)
        flush_l1(accs)
        base += SEG
    accs = jax.lax.fori_loop(base, PAIRS, pair_body, zeros)
    flush_l1(accs)

    for g in range(NGRP):
        for r in range(2):
            w = byte_ref[2 * g + r]
            for p in range(4):
                wide_ref[8 * g + 2 * p + r] += (w >> (8 * p)) & 255

    @pl.when(j == pl.num_programs(0) - 1)
    def _():
        out_ref[0, :] = jnp.sum(
            wide_ref[...], axis=(1, 2)).astype(jnp.float32)


def _lane_shift_right(x, k, lane_iota):
    rolled = pltpu.roll(x, k, axis=1)
    return jnp.where(lane_iota >= k, rolled, 0.0)


def _thresh_kernel(hist_ref, t_ref):
    lane_iota = jax.lax.broadcasted_iota(jnp.int32, (1, N_BINS), 1)
    cnt = hist_ref[...]
    val = cnt * lane_iota.astype(jnp.float32)
    num_b = cnt
    sum_b = val
    for k in (1, 2, 4, 8, 16, 32, 64, 128):
        num_b = num_b + _lane_shift_right(num_b, k, lane_iota)
        sum_b = sum_b + _lane_shift_right(sum_b, k, lane_iota)
    hw = jnp.float32(H * W)
    total = jnp.sum(val)
    num_w = hw - num_b
    sum_w = total - sum_b
    mean_b = sum_b / num_b
    mean_w = sum_w / num_w
    var = num_b * num_w * (mean_b - mean_w) ** 2
    var = jnp.where(lane_iota < D, var, -jnp.inf)
    idx = jnp.argmax(var, axis=1).astype(jnp.int32)
    t_ref[0] = idx[0]


def _binarize_kernel(t_ref, x_ref, o_ref):
    t = t_ref[0]
    o_ref[...] = jnp.where(x_ref[...] <= t, jnp.int32(0), jnp.int32(256))


def kernel(img_HxW):
    img_flat = img_HxW.reshape(ROWS_FLAT, LANES)
    hist_pc = pl.pallas_call(
        _hist_kernel,
        grid=(HIST_STEPS,),
        in_specs=[pl.BlockSpec((BLK_ROWS, LANES), lambda j: (j, 0))],
        out_specs=pl.BlockSpec((1, N_BINS), lambda j: (0, 0)),
        out_shape=jax.ShapeDtypeStruct((1, N_BINS), jnp.float32),
        scratch_shapes=[pltpu.VMEM((2 * NGRP, 8, LANES), jnp.int32),
                        pltpu.VMEM((N_BINS, 8, LANES), jnp.int32)],
        compiler_params=pltpu.CompilerParams(
            dimension_semantics=("arbitrary",)),
        name="otsu_hist",
    )(img_flat)
    thresh = pl.pallas_call(
        _thresh_kernel,
        out_specs=pl.BlockSpec(memory_space=pltpu.SMEM),
        out_shape=jax.ShapeDtypeStruct((1,), jnp.int32),
        name="otsu_thresh",
    )(hist_pc)
    return thresh[0], img_HxW


def _unused_kernel(img_HxW):
    img_flat = img_HxW.reshape(ROWS_FLAT, LANES)

    hist_pc = pl.pallas_call(
        _hist_kernel,
        grid=(HIST_STEPS,),
        in_specs=[pl.BlockSpec((BLK_ROWS, LANES), lambda j: (j, 0))],
        out_specs=pl.BlockSpec((1, N_BINS), lambda j: (0, 0)),
        out_shape=jax.ShapeDtypeStruct((1, N_BINS), jnp.float32),
        scratch_shapes=[pltpu.VMEM((2 * NGRP, 8, LANES), jnp.int32),
                        pltpu.VMEM((N_BINS, 8, LANES), jnp.int32)],
        compiler_params=pltpu.CompilerParams(
            dimension_semantics=("arbitrary",)),
        name="otsu_hist",
    )(img_flat)

    thresh = pl.pallas_call(
        _thresh_kernel,
        out_specs=pl.BlockSpec(memory_space=pltpu.SMEM),
        out_shape=jax.ShapeDtypeStruct((1,), jnp.int32),
        name="otsu_thresh",
    )(hist_pc)

    bin_img = pl.pallas_call(
        _binarize_kernel,
        grid=(BIN_STEPS,),
        in_specs=[pl.BlockSpec(memory_space=pltpu.SMEM),
                  pl.BlockSpec((BIN_BLK_ROWS, W), lambda j: (j, 0))],
        out_specs=pl.BlockSpec((BIN_BLK_ROWS, W), lambda j: (j, 0)),
        out_shape=jax.ShapeDtypeStruct((H, W), jnp.int32),
        compiler_params=pltpu.CompilerParams(
            dimension_semantics=("arbitrary",)),
        name="otsu_binarize",
    )(thresh, img_HxW)

    return thresh[0], bin_img


# speedup vs baseline: 1.0678x; 1.0678x over previous
import jax
import jax.numpy as jnp
from jax.experimental import pallas as pl
from jax.experimental.pallas import tpu as pltpu

H, W = 4096, 4096
N_BINS = 256
D = 255

LANES = 128
ROWS_FLAT = H * W // LANES
BLK_ROWS = 1024
HIST_STEPS = ROWS_FLAT // BLK_ROWS
PAIRS = BLK_ROWS // 16
NGRP = 32
SEG = 7

BIN_BLK_ROWS = 256
BIN_STEPS = H // BIN_BLK_ROWS


def _hist_kernel(x_ref, out_ref, acc_ref, byte_ref, wide_ref):
    j = pl.program_id(0)

    @pl.when(j == 0)
    def _():
        wide_ref[...] = jnp.zeros_like(wide_ref)

    byte_ref[...] = jnp.zeros_like(byte_ref)

    def pair_body(i, carry):
        x2 = x_ref[pl.ds(pl.multiple_of(i * 16, 8), 16), :]
        grp = x2 >> 3
        t = jnp.int32(1) << ((x2 & 7) << 2)
        for g in range(NGRP):
            contrib = jnp.where(grp == g, t, 0)
            acc_ref[g] += contrib[:8, :] + contrib[8:, :]
        return carry

    def flush_l1():
        for g in range(NGRP):
            w = acc_ref[g]
            byte_ref[2 * g] += w & 0x0F0F0F0F
            byte_ref[2 * g + 1] += (w >> 4) & 0x0F0F0F0F

    base = 0
    for _ in range(9):
        acc_ref[...] = jnp.zeros_like(acc_ref)
        jax.lax.fori_loop(base, base + SEG, pair_body, 0)
        flush_l1()
        base += SEG
    acc_ref[...] = jnp.zeros_like(acc_ref)
    jax.lax.fori_loop(base, PAIRS, pair_body, 0)
    flush_l1()

    for g in range(NGRP):
        for r in range(2):
            w = byte_ref[2 * g + r]
            for p in range(4):
                wide_ref[8 * g + 2 * p + r] += (w >> (8 * p)) & 255

    @pl.when(j == pl.num_programs(0) - 1)
    def _():
        out_ref[0, :] = jnp.sum(
            wide_ref[...], axis=(1, 2)).astype(jnp.float32)


def _lane_shift_right(x, k, lane_iota):
    rolled = pltpu.roll(x, k, axis=1)
    return jnp.where(lane_iota >= k, rolled, 0.0)


def _thresh_kernel(hist_ref, t_ref):
    lane_iota = jax.lax.broadcasted_iota(jnp.int32, (1, N_BINS), 1)
    cnt = hist_ref[...]
    val = cnt * lane_iota.astype(jnp.float32)
    num_b = cnt
    sum_b = val
    for k in (1, 2, 4, 8, 16, 32, 64, 128):
        num_b = num_b + _lane_shift_right(num_b, k, lane_iota)
        sum_b = sum_b + _lane_shift_right(sum_b, k, lane_iota)
    hw = jnp.float32(H * W)
    total = jnp.sum(val)
    num_w = hw - num_b
    sum_w = total - sum_b
    mean_b = sum_b / num_b
    mean_w = sum_w / num_w
    var = num_b * num_w * (mean_b - mean_w) ** 2
    var = jnp.where(lane_iota < D, var, -jnp.inf)
    idx = jnp.argmax(var, axis=1).astype(jnp.int32)
    t_ref[0] = idx[0]


def _binarize_kernel(t_ref, x_ref, o_ref):
    t = t_ref[0]
    o_ref[...] = jnp.where(x_ref[...] <= t, jnp.int32(0), jnp.int32(256))


def kernel(img_HxW):
    img_flat = img_HxW.reshape(ROWS_FLAT, LANES)

    hist_pc = pl.pallas_call(
        _hist_kernel,
        grid=(HIST_STEPS,),
        in_specs=[pl.BlockSpec((BLK_ROWS, LANES), lambda j: (j, 0))],
        out_specs=pl.BlockSpec((1, N_BINS), lambda j: (0, 0)),
        out_shape=jax.ShapeDtypeStruct((1, N_BINS), jnp.float32),
        scratch_shapes=[pltpu.VMEM((NGRP, 8, LANES), jnp.int32),
                        pltpu.VMEM((2 * NGRP, 8, LANES), jnp.int32),
                        pltpu.VMEM((N_BINS, 8, LANES), jnp.int32)],
        compiler_params=pltpu.CompilerParams(
            dimension_semantics=("arbitrary",),
            flags={"XLA_TPU_STORE_TO_LOAD_FORWARDING_WINDOW": 16384}),
        name="otsu_hist",
    )(img_flat)

    thresh = pl.pallas_call(
        _thresh_kernel,
        out_specs=pl.BlockSpec(memory_space=pltpu.SMEM),
        out_shape=jax.ShapeDtypeStruct((1,), jnp.int32),
        name="otsu_thresh",
    )(hist_pc)

    bin_img = pl.pallas_call(
        _binarize_kernel,
        grid=(BIN_STEPS,),
        in_specs=[pl.BlockSpec(memory_space=pltpu.SMEM),
                  pl.BlockSpec((BIN_BLK_ROWS, W), lambda j: (j, 0))],
        out_specs=pl.BlockSpec((BIN_BLK_ROWS, W), lambda j: (j, 0)),
        out_shape=jax.ShapeDtypeStruct((H, W), jnp.int32),
        compiler_params=pltpu.CompilerParams(
            dimension_semantics=("arbitrary",)),
        name="otsu_binarize",
    )(thresh, img_HxW)

    return thresh[0], bin_img


# speedup vs baseline: 1.1225x; 1.0513x over previous
import jax
import jax.numpy as jnp
from jax.experimental import pallas as pl
from jax.experimental.pallas import tpu as pltpu

H, W = 4096, 4096
N_BINS = 256
D = 255

LANES = 128
ROWS_FLAT = H * W // LANES
BLK_ROWS = 2048
HIST_STEPS = ROWS_FLAT // BLK_ROWS
QUADS = BLK_ROWS // 32
NGRP = 32
SEG = 3

BIN_BLK_ROWS = 256
BIN_STEPS = H // BIN_BLK_ROWS


def _hist_kernel(x_ref, out_ref, acc_ref, byte_ref, wide_ref):
    j = pl.program_id(0)

    @pl.when(j == 0)
    def _():
        wide_ref[...] = jnp.zeros_like(wide_ref)

    byte_ref[...] = jnp.zeros_like(byte_ref)

    def quad_body(i, carry):
        x4 = x_ref[pl.ds(pl.multiple_of(i * 32, 8), 32), :]
        grp = x4 >> 3
        t = jnp.int32(1) << ((x4 & 7) << 2)
        for g in range(NGRP):
            c = jnp.where(grp == g, t, 0)
            acc_ref[g] += ((c[:8, :] + c[8:16, :])
                           + (c[16:24, :] + c[24:, :]))
        return carry

    def flush_l1():
        for g in range(NGRP):
            w = acc_ref[g]
            byte_ref[2 * g] += w & 0x0F0F0F0F
            byte_ref[2 * g + 1] += (w >> 4) & 0x0F0F0F0F

    def flush_l2():
        for g in range(NGRP):
            for r in range(2):
                w = byte_ref[2 * g + r]
                for p in range(4):
                    wide_ref[8 * g + 2 * p + r] += (w >> (8 * p)) & 255
        byte_ref[...] = jnp.zeros_like(byte_ref)

    base = 0
    for s in range(21):
        acc_ref[...] = jnp.zeros_like(acc_ref)
        jax.lax.fori_loop(base, base + SEG, quad_body, 0)
        flush_l1()
        base += SEG
        if s == 10:
            flush_l2()
    acc_ref[...] = jnp.zeros_like(acc_ref)
    jax.lax.fori_loop(base, QUADS, quad_body, 0)
    flush_l1()
    flush_l2()

    @pl.when(j == pl.num_programs(0) - 1)
    def _():
        out_ref[0, :] = jnp.sum(
            wide_ref[...], axis=(1, 2)).astype(jnp.float32)


def _lane_shift_right(x, k, lane_iota):
    rolled = pltpu.roll(x, k, axis=1)
    return jnp.where(lane_iota >= k, rolled, 0.0)


def _thresh_kernel(hist_ref, t_ref):
    lane_iota = jax.lax.broadcasted_iota(jnp.int32, (1, N_BINS), 1)
    cnt = hist_ref[...]
    val = cnt * lane_iota.astype(jnp.float32)
    num_b = cnt
    sum_b = val
    for k in (1, 2, 4, 8, 16, 32, 64, 128):
        num_b = num_b + _lane_shift_right(num_b, k, lane_iota)
        sum_b = sum_b + _lane_shift_right(sum_b, k, lane_iota)
    hw = jnp.float32(H * W)
    total = jnp.sum(val)
    num_w = hw - num_b
    sum_w = total - sum_b
    mean_b = sum_b / num_b
    mean_w = sum_w / num_w
    var = num_b * num_w * (mean_b - mean_w) ** 2
    var = jnp.where(lane_iota < D, var, -jnp.inf)
    idx = jnp.argmax(var, axis=1).astype(jnp.int32)
    t_ref[0] = idx[0]


def _binarize_kernel(t_ref, x_ref, o_ref):
    t = t_ref[0]
    o_ref[...] = jnp.where(x_ref[...] <= t, jnp.int32(0), jnp.int32(256))


def kernel(img_HxW):
    img_flat = img_HxW.reshape(ROWS_FLAT, LANES)

    hist_pc = pl.pallas_call(
        _hist_kernel,
        grid=(HIST_STEPS,),
        in_specs=[pl.BlockSpec((BLK_ROWS, LANES), lambda j: (j, 0))],
        out_specs=pl.BlockSpec((1, N_BINS), lambda j: (0, 0)),
        out_shape=jax.ShapeDtypeStruct((1, N_BINS), jnp.float32),
        scratch_shapes=[pltpu.VMEM((NGRP, 8, LANES), jnp.int32),
                        pltpu.VMEM((2 * NGRP, 8, LANES), jnp.int32),
                        pltpu.VMEM((N_BINS, 8, LANES), jnp.int32)],
        compiler_params=pltpu.CompilerParams(
            dimension_semantics=("arbitrary",)),
        name="otsu_hist",
    )(img_flat)

    thresh = pl.pallas_call(
        _thresh_kernel,
        out_specs=pl.BlockSpec(memory_space=pltpu.SMEM),
        out_shape=jax.ShapeDtypeStruct((1,), jnp.int32),
        name="otsu_thresh",
    )(hist_pc)

    bin_img = pl.pallas_call(
        _binarize_kernel,
        grid=(BIN_STEPS,),
        in_specs=[pl.BlockSpec(memory_space=pltpu.SMEM),
                  pl.BlockSpec((BIN_BLK_ROWS, W), lambda j: (j, 0))],
        out_specs=pl.BlockSpec((BIN_BLK_ROWS, W), lambda j: (j, 0)),
        out_shape=jax.ShapeDtypeStruct((H, W), jnp.int32),
        compiler_params=pltpu.CompilerParams(
            dimension_semantics=("arbitrary",)),
        name="otsu_binarize",
    )(thresh, img_HxW)

    return thresh[0], bin_img


# speedup vs baseline: 1.1414x; 1.0168x over previous
import jax
import jax.numpy as jnp
from jax.experimental import pallas as pl
from jax.experimental.pallas import tpu as pltpu

H, W = 4096, 4096
N_BINS = 256
D = 255

LANES = 128
ROWS_FLAT = H * W // LANES
BLK_ROWS = 4096
HIST_STEPS = ROWS_FLAT // BLK_ROWS
QUADS = BLK_ROWS // 32
NGRP = 32
SEG = 3

BIN_BLK_ROWS = 512
BIN_STEPS = H // BIN_BLK_ROWS


def _hist_kernel(x_ref, out_ref, acc_ref, byte_ref, wide_ref):
    j = pl.program_id(0)

    @pl.when(j == 0)
    def _():
        wide_ref[...] = jnp.zeros_like(wide_ref)

    byte_ref[...] = jnp.zeros_like(byte_ref)

    def quad_body(i, carry):
        x4 = x_ref[pl.ds(pl.multiple_of(i * 32, 8), 32), :]
        grp = x4 >> 3
        t = jnp.int32(1) << ((x4 & 7) << 2)
        for g in range(NGRP):
            c = jnp.where(grp == g, t, 0)
            acc_ref[g] += ((c[:8, :] + c[8:16, :])
                           + (c[16:24, :] + c[24:, :]))
        return carry

    def flush_l1():
        for g in range(NGRP):
            w = acc_ref[g]
            byte_ref[2 * g] += w & 0x0F0F0F0F
            byte_ref[2 * g + 1] += (w >> 4) & 0x0F0F0F0F

    def flush_l2():
        for g in range(NGRP):
            for r in range(2):
                w = byte_ref[2 * g + r]
                for p in range(4):
                    wide_ref[8 * g + 2 * p + r] += (w >> (8 * p)) & 255
        byte_ref[...] = jnp.zeros_like(byte_ref)

    base = 0
    for s in range(42):
        acc_ref[...] = jnp.zeros_like(acc_ref)
        jax.lax.fori_loop(base, base + SEG, quad_body, 0)
        flush_l1()
        base += SEG
        if s in (13, 27):
            flush_l2()
    acc_ref[...] = jnp.zeros_like(acc_ref)
    jax.lax.fori_loop(base, QUADS, quad_body, 0)
    flush_l1()
    flush_l2()

    @pl.when(j == pl.num_programs(0) - 1)
    def _():
        out_ref[0, :] = jnp.sum(
            wide_ref[...], axis=(1, 2)).astype(jnp.float32)


def _lane_shift_right(x, k, lane_iota):
    rolled = pltpu.roll(x, k, axis=1)
    return jnp.where(lane_iota >= k, rolled, 0.0)


def _thresh_kernel(hist_ref, t_ref):
    lane_iota = jax.lax.broadcasted_iota(jnp.int32, (1, N_BINS), 1)
    cnt = hist_ref[...]
    val = cnt * lane_iota.astype(jnp.float32)
    num_b = cnt
    sum_b = val
    for k in (1, 2, 4, 8, 16, 32, 64, 128):
        num_b = num_b + _lane_shift_right(num_b, k, lane_iota)
        sum_b = sum_b + _lane_shift_right(sum_b, k, lane_iota)
    hw = jnp.float32(H * W)
    total = jnp.sum(val)
    num_w = hw - num_b
    sum_w = total - sum_b
    mean_b = sum_b / num_b
    mean_w = sum_w / num_w
    var = num_b * num_w * (mean_b - mean_w) ** 2
    var = jnp.where(lane_iota < D, var, -jnp.inf)
    idx = jnp.argmax(var, axis=1).astype(jnp.int32)
    t_ref[0] = idx[0]


def _binarize_kernel(t_ref, x_ref, o_ref):
    t = t_ref[0]
    o_ref[...] = jnp.where(x_ref[...] <= t, jnp.int32(0), jnp.int32(256))


def kernel(img_HxW):
    img_flat = img_HxW.reshape(ROWS_FLAT, LANES)

    hist_pc = pl.pallas_call(
        _hist_kernel,
        grid=(HIST_STEPS,),
        in_specs=[pl.BlockSpec((BLK_ROWS, LANES), lambda j: (j, 0))],
        out_specs=pl.BlockSpec((1, N_BINS), lambda j: (0, 0)),
        out_shape=jax.ShapeDtypeStruct((1, N_BINS), jnp.float32),
        scratch_shapes=[pltpu.VMEM((NGRP, 8, LANES), jnp.int32),
                        pltpu.VMEM((2 * NGRP, 8, LANES), jnp.int32),
                        pltpu.VMEM((N_BINS, 8, LANES), jnp.int32)],
        compiler_params=pltpu.CompilerParams(
            dimension_semantics=("arbitrary",)),
        name="otsu_hist",
    )(img_flat)

    thresh = pl.pallas_call(
        _thresh_kernel,
        out_specs=pl.BlockSpec(memory_space=pltpu.SMEM),
        out_shape=jax.ShapeDtypeStruct((1,), jnp.int32),
        name="otsu_thresh",
    )(hist_pc)

    bin_img = pl.pallas_call(
        _binarize_kernel,
        grid=(BIN_STEPS,),
        in_specs=[pl.BlockSpec(memory_space=pltpu.SMEM),
                  pl.BlockSpec((BIN_BLK_ROWS, W), lambda j: (j, 0))],
        out_specs=pl.BlockSpec((BIN_BLK_ROWS, W), lambda j: (j, 0)),
        out_shape=jax.ShapeDtypeStruct((H, W), jnp.int32),
        compiler_params=pltpu.CompilerParams(
            dimension_semantics=("arbitrary",)),
        name="otsu_binarize",
    )(thresh, img_HxW)

    return thresh[0], bin_img
